# baseline (speedup 1.0000x reference)
import functools

import jax
import jax.numpy as jnp
from jax import lax
from jax.experimental import pallas as pl
from jax.experimental.pallas import tpu as pltpu
from jax.experimental.pallas import tpu_sc as plsc

N, D, B, P, NB, SF, CF = 10000, 128, 1024, 6, 32, 128, 128
PB = P * B
NC, NS = 2, 16
NW = NC * NS
PAIRS_W = PB // NW
GROUP = 4
GROUPS_W = PAIRS_W // GROUP
ROWS_G = GROUP * NB


def _xw_body(x_ref, w_ref, b_ref, o_ref):
    o_ref[...] = jnp.maximum(
        jnp.dot(x_ref[...], w_ref[...], preferred_element_type=jnp.float32)
        + b_ref[...], 0.0).astype(jnp.bfloat16)


def _sc_body(xw_hbm, x_hbm, nidx_hbm, pidx_hbm, hmax_out, self_out,
             nidx_v, pidx_v, nbuf0, nbuf1, selfbuf, hbuf, sem0, sem1, sem2):
    wid = lax.axis_index("s") * NC + lax.axis_index("c")
    gbase = wid * GROUPS_W
    rbase = wid * PAIRS_W

    pltpu.sync_copy(nidx_hbm.at[pl.ds(gbase, GROUPS_W)], nidx_v)
    pltpu.sync_copy(pidx_hbm.at[pl.ds(wid * 2, 2)], pidx_v)

    pltpu.async_copy(x_hbm.at[pidx_v.at[0]], selfbuf.at[pl.ds(0, 96)], sem2)
    pltpu.async_copy(x_hbm.at[pidx_v.at[1]], selfbuf.at[pl.ds(96, 96)], sem2)

    def start(g, buf, sem):
        pltpu.async_copy(xw_hbm.at[nidx_v.at[g]], buf, sem)

    def wait(buf, sem):
        pltpu.make_async_copy(xw_hbm.at[nidx_v.at[0]], buf, sem).wait()

    hi_mask = jnp.int32(-65536)
    shr16 = jnp.full((16,), 16, jnp.int32)

    def reduce_group(buf, g):
        for q in range(GROUP):
            r0 = q * NB
            for c in range(D // 32):
                v = buf[r0, pl.ds(c * 16, 16)]
                acc_hi = v & hi_mask
                acc_lo = v << 16
                for r in range(1, NB):
                    v = buf[r0 + r, pl.ds(c * 16, 16)]
                    acc_hi = jnp.maximum(acc_hi, v & hi_mask)
                    acc_lo = jnp.maximum(acc_lo, v << 16)
                hbuf[g, q, pl.ds(c * 16, 16)] = acc_hi | (
                    jax.lax.shift_right_logical(acc_lo, shr16))

    start(0, nbuf0, sem0)

    def mbody(m, carry):
        g0 = 2 * m
        start(g0 + 1, nbuf1, sem1)
        wait(nbuf0, sem0)
        reduce_group(nbuf0, g0)

        @pl.when(m < GROUPS_W // 2 - 1)
        def _():
            start(g0 + 2, nbuf0, sem0)

        wait(nbuf1, sem1)
        reduce_group(nbuf1, g0 + 1)
        return carry

    lax.fori_loop(0, GROUPS_W // 2, mbody, 0, unroll=False)

    pltpu.make_async_copy(x_hbm.at[pidx_v.at[0]],
                          selfbuf.at[pl.ds(0, 96)], sem2).wait()
    pltpu.make_async_copy(x_hbm.at[pidx_v.at[1]],
                          selfbuf.at[pl.ds(96, 96)], sem2).wait()

    pltpu.sync_copy(hbuf, hmax_out.at[pl.ds(gbase, GROUPS_W)])
    pltpu.sync_copy(selfbuf, self_out.at[pl.ds(rbase, PAIRS_W)])


_sc_gather_max = functools.partial(
    pl.kernel,
    out_type=(jax.ShapeDtypeStruct((PB // GROUP, GROUP, D // 2), jnp.int32),
              jax.ShapeDtypeStruct((PB, D), jnp.float32)),
    mesh=plsc.VectorSubcoreMesh(core_axis_name="c", subcore_axis_name="s"),
    compiler_params=pltpu.CompilerParams(use_tc_tiling_on_sc=False),
    scratch_types=[
        pltpu.VMEM((GROUPS_W, ROWS_G), jnp.int32),
        pltpu.VMEM((2, 96), jnp.int32),
        pltpu.VMEM((ROWS_G, D // 2), jnp.int32),
        pltpu.VMEM((ROWS_G, D // 2), jnp.int32),
        pltpu.VMEM((PAIRS_W, D), jnp.float32),
        pltpu.VMEM((GROUPS_W, GROUP, D // 2), jnp.int32),
        pltpu.SemaphoreType.DMA,
        pltpu.SemaphoreType.DMA,
        pltpu.SemaphoreType.DMA,
    ])(_sc_body)


def _head_body(self_ref, hmax_ref, was_ref, wah_ref, bagg_ref,
               k20_ref, k21_ref, bk2_ref, k30_ref, k31_ref, k32_ref, bk3_ref,
               w1a_ref, w1b_ref, b1_ref, w2_ref, b2_ref, w3_ref, b3_ref,
               o_ref):
    def dot(a, b):
        return jnp.dot(a, b, preferred_element_type=jnp.float32)

    steps = []
    for t in range(P):
        s = (dot(self_ref[t], was_ref[...])
             + dot(hmax_ref[t].astype(jnp.float32), wah_ref[...])
             + bagg_ref[...])
        steps.append(jnp.maximum(s, 0.0))

    p2 = None
    for t in range(P - 1):
        c = jnp.maximum(dot(steps[t], k20_ref[...])
                        + dot(steps[t + 1], k21_ref[...]) + bk2_ref[...], 0.0)
        p2 = c if p2 is None else jnp.maximum(p2, c)

    p3 = None
    for t in range(P - 2):
        c = jnp.maximum(dot(steps[t], k30_ref[...])
                        + dot(steps[t + 1], k31_ref[...])
                        + dot(steps[t + 2], k32_ref[...]) + bk3_ref[...], 0.0)
        p3 = c if p3 is None else jnp.maximum(p3, c)

    h1 = jnp.maximum(dot(p2, w1a_ref[...]) + dot(p3, w1b_ref[...])
                     + b1_ref[...], 0.0)
    h2 = jnp.maximum(dot(h1, w2_ref[...]) + b2_ref[...], 0.0)
    logits = dot(h2, w3_ref[...]) + b3_ref[...]
    mx = jnp.max(logits, axis=-1, keepdims=True)
    e = jnp.exp(logits - mx)
    o_ref[...] = e / jnp.sum(e, axis=-1, keepdims=True)


def kernel(x, path_nodes, neighbor_idx, W_pool, b_pool, W_agg, b_agg,
           K2, bK2, K3, bK3, W1, b1, W2, b2, W3, b3):
    xw = pl.pallas_call(
        _xw_body,
        grid=(10,),
        in_specs=[pl.BlockSpec((N // 10, D), lambda i: (i, 0)),
                  pl.BlockSpec((D, D), lambda i: (0, 0)),
                  pl.BlockSpec((1, D), lambda i: (0, 0))],
        out_specs=pl.BlockSpec((N // 10, D), lambda i: (i, 0)),
        out_shape=jax.ShapeDtypeStruct((N, D), jnp.bfloat16),
    )(x, W_pool, b_pool.reshape(1, D))

    xw_pack = jax.lax.bitcast_convert_type(xw.reshape(N, D // 2, 2), jnp.int32)

    nidx_t = jnp.transpose(neighbor_idx, (1, 0, 2)).reshape(PB * NB // ROWS_G,
                                                            ROWS_G)
    pidx_t = jnp.transpose(path_nodes, (1, 0)).reshape(NW * 2, 96)

    hmax_pack, self_flat = _sc_gather_max(xw_pack, x, nidx_t, pidx_t)
    hmax_flat = jax.lax.bitcast_convert_type(hmax_pack, jnp.bfloat16)

    BBLK = 256
    full = lambda shape: pl.BlockSpec(shape, lambda i: tuple(0 for _ in shape))
    out = pl.pallas_call(
        _head_body,
        grid=(B // BBLK,),
        in_specs=[
            pl.BlockSpec((P, BBLK, D), lambda i: (0, i, 0)),
            pl.BlockSpec((P, BBLK, D), lambda i: (0, i, 0)),
            full((SF, SF)), full((SF, SF)), full((1, SF)),
            full((SF, CF)), full((SF, CF)), full((1, CF)),
            full((SF, CF)), full((SF, CF)), full((SF, CF)), full((1, CF)),
            full((CF, 400)), full((CF, 400)), full((1, 400)),
            full((400, 400)), full((1, 400)),
            full((400, 2)), full((1, 2)),
        ],
        out_specs=pl.BlockSpec((BBLK, 2), lambda i: (i, 0)),
        out_shape=jax.ShapeDtypeStruct((B, 2), jnp.float32),
    )(self_flat.reshape(P, B, D), hmax_flat.reshape(P, B, D),
      W_agg[:D], W_agg[D:], b_agg.reshape(1, SF),
      K2[0], K2[1], bK2.reshape(1, CF),
      K3[0], K3[1], K3[2], bK3.reshape(1, CF),
      W1[:CF], W1[CF:], b1.reshape(1, 400),
      W2, b2.reshape(1, 400),
      W3, b3.reshape(1, 2))
    return out


# speedup vs baseline: 2.3108x; 2.3108x over previous
import functools

import jax
import jax.numpy as jnp
from jax import lax
from jax.experimental import pallas as pl
from jax.experimental.pallas import tpu as pltpu
from jax.experimental.pallas import tpu_sc as plsc

N, D, B, P, NB, SF, CF = 10000, 128, 1024, 6, 32, 128, 128
PB = P * B
NC, NS = 2, 16
NW = NC * NS
PAIRS_W = PB // NW
GROUP = 4
GROUPS_W = PAIRS_W // GROUP
ROWS_G = GROUP * NB


def _xw_body(x_ref, w_ref, b_ref, o_ref):
    o_ref[...] = jnp.maximum(
        jnp.dot(x_ref[...], w_ref[...], preferred_element_type=jnp.float32)
        + b_ref[...], 0.0)


def _sc_body(xw_hbm, x_hbm, nidx_hbm, pidx_hbm, hmax_out, self_out,
             nidx_v, pidx_v, nbuf0, nbuf1, selfbuf, hbuf, sem0, sem1, sem2):
    wid = lax.axis_index("s") * NC + lax.axis_index("c")
    gbase = wid * GROUPS_W
    rbase = wid * PAIRS_W

    pltpu.sync_copy(nidx_hbm.at[pl.ds(gbase, GROUPS_W)], nidx_v)
    pltpu.sync_copy(pidx_hbm.at[pl.ds(wid * 2, 2)], pidx_v)

    pltpu.async_copy(x_hbm.at[pidx_v.at[0]], selfbuf.at[pl.ds(0, 96)], sem2)
    pltpu.async_copy(x_hbm.at[pidx_v.at[1]], selfbuf.at[pl.ds(96, 96)], sem2)

    def start(g, buf, sem):
        pltpu.async_copy(xw_hbm.at[nidx_v.at[g]], buf, sem)

    def wait(buf, sem):
        pltpu.make_async_copy(xw_hbm.at[nidx_v.at[0]], buf, sem).wait()

    def reduce_group(buf, g):
        for q in range(GROUP):
            r0 = q * NB
            for c in range(D // 16):
                acc = buf[r0, pl.ds(c * 16, 16)]
                for r in range(1, NB):
                    acc = jnp.maximum(acc, buf[r0 + r, pl.ds(c * 16, 16)])
                hbuf[g, q, pl.ds(c * 16, 16)] = acc

    start(0, nbuf0, sem0)

    def mbody(m, carry):
        g0 = 2 * m
        start(g0 + 1, nbuf1, sem1)
        wait(nbuf0, sem0)
        reduce_group(nbuf0, g0)

        @pl.when(m < GROUPS_W // 2 - 1)
        def _():
            start(g0 + 2, nbuf0, sem0)

        wait(nbuf1, sem1)
        reduce_group(nbuf1, g0 + 1)
        return carry

    lax.fori_loop(0, GROUPS_W // 2, mbody, 0, unroll=False)

    pltpu.make_async_copy(x_hbm.at[pidx_v.at[0]],
                          selfbuf.at[pl.ds(0, 96)], sem2).wait()
    pltpu.make_async_copy(x_hbm.at[pidx_v.at[1]],
                          selfbuf.at[pl.ds(96, 96)], sem2).wait()

    pltpu.sync_copy(hbuf, hmax_out.at[pl.ds(gbase, GROUPS_W)])
    pltpu.sync_copy(selfbuf, self_out.at[pl.ds(rbase, PAIRS_W)])


_sc_gather_max = functools.partial(
    pl.kernel,
    out_type=(jax.ShapeDtypeStruct((PB // GROUP, GROUP, D), jnp.float32),
              jax.ShapeDtypeStruct((PB, D), jnp.float32)),
    mesh=plsc.VectorSubcoreMesh(core_axis_name="c", subcore_axis_name="s"),
    compiler_params=pltpu.CompilerParams(use_tc_tiling_on_sc=False),
    scratch_types=[
        pltpu.VMEM((GROUPS_W, ROWS_G), jnp.int32),
        pltpu.VMEM((2, 96), jnp.int32),
        pltpu.VMEM((ROWS_G, D), jnp.float32),
        pltpu.VMEM((ROWS_G, D), jnp.float32),
        pltpu.VMEM((PAIRS_W, D), jnp.float32),
        pltpu.VMEM((GROUPS_W, GROUP, D), jnp.float32),
        pltpu.SemaphoreType.DMA,
        pltpu.SemaphoreType.DMA,
        pltpu.SemaphoreType.DMA,
    ])(_sc_body)


def _head_body(self_ref, hmax_ref, was_ref, wah_ref, bagg_ref,
               k20_ref, k21_ref, bk2_ref, k30_ref, k31_ref, k32_ref, bk3_ref,
               w1a_ref, w1b_ref, b1_ref, w2_ref, b2_ref, w3_ref, b3_ref,
               o_ref):
    def dot(a, b):
        return jnp.dot(a, b, preferred_element_type=jnp.float32)

    steps = []
    for t in range(P):
        s = (dot(self_ref[t], was_ref[...]) + dot(hmax_ref[t], wah_ref[...])
             + bagg_ref[...])
        steps.append(jnp.maximum(s, 0.0))

    p2 = None
    for t in range(P - 1):
        c = jnp.maximum(dot(steps[t], k20_ref[...])
                        + dot(steps[t + 1], k21_ref[...]) + bk2_ref[...], 0.0)
        p2 = c if p2 is None else jnp.maximum(p2, c)

    p3 = None
    for t in range(P - 2):
        c = jnp.maximum(dot(steps[t], k30_ref[...])
                        + dot(steps[t + 1], k31_ref[...])
                        + dot(steps[t + 2], k32_ref[...]) + bk3_ref[...], 0.0)
        p3 = c if p3 is None else jnp.maximum(p3, c)

    h1 = jnp.maximum(dot(p2, w1a_ref[...]) + dot(p3, w1b_ref[...])
                     + b1_ref[...], 0.0)
    h2 = jnp.maximum(dot(h1, w2_ref[...]) + b2_ref[...], 0.0)
    logits = dot(h2, w3_ref[...]) + b3_ref[...]
    mx = jnp.max(logits, axis=-1, keepdims=True)
    e = jnp.exp(logits - mx)
    o_ref[...] = e / jnp.sum(e, axis=-1, keepdims=True)


def kernel(x, path_nodes, neighbor_idx, W_pool, b_pool, W_agg, b_agg,
           K2, bK2, K3, bK3, W1, b1, W2, b2, W3, b3):
    xw = pl.pallas_call(
        _xw_body,
        grid=(10,),
        in_specs=[pl.BlockSpec((N // 10, D), lambda i: (i, 0)),
                  pl.BlockSpec((D, D), lambda i: (0, 0)),
                  pl.BlockSpec((1, D), lambda i: (0, 0))],
        out_specs=pl.BlockSpec((N // 10, D), lambda i: (i, 0)),
        out_shape=jax.ShapeDtypeStruct((N, D), jnp.float32),
    )(x, W_pool, b_pool.reshape(1, D))

    nidx_t = jnp.transpose(neighbor_idx, (1, 0, 2)).reshape(PB * NB // ROWS_G,
                                                            ROWS_G)
    pidx_t = jnp.transpose(path_nodes, (1, 0)).reshape(NW * 2, 96)

    hmax_flat, self_flat = _sc_gather_max(xw, x, nidx_t, pidx_t)

    BBLK = 256
    full = lambda shape: pl.BlockSpec(shape, lambda i: tuple(0 for _ in shape))
    out = pl.pallas_call(
        _head_body,
        grid=(B // BBLK,),
        in_specs=[
            pl.BlockSpec((P, BBLK, D), lambda i: (0, i, 0)),
            pl.BlockSpec((P, BBLK, D), lambda i: (0, i, 0)),
            full((SF, SF)), full((SF, SF)), full((1, SF)),
            full((SF, CF)), full((SF, CF)), full((1, CF)),
            full((SF, CF)), full((SF, CF)), full((SF, CF)), full((1, CF)),
            full((CF, 400)), full((CF, 400)), full((1, 400)),
            full((400, 400)), full((1, 400)),
            full((400, 2)), full((1, 2)),
        ],
        out_specs=pl.BlockSpec((BBLK, 2), lambda i: (i, 0)),
        out_shape=jax.ShapeDtypeStruct((B, 2), jnp.float32),
    )(self_flat.reshape(P, B, D), hmax_flat.reshape(P, B, D),
      W_agg[:D], W_agg[D:], b_agg.reshape(1, SF),
      K2[0], K2[1], bK2.reshape(1, CF),
      K3[0], K3[1], K3[2], bK3.reshape(1, CF),
      W1[:CF], W1[CF:], b1.reshape(1, 400),
      W2, b2.reshape(1, 400),
      W3, b3.reshape(1, 2))
    return out


# speedup vs baseline: 3.3527x; 1.4509x over previous
import functools

import jax
import jax.numpy as jnp
from jax import lax
from jax.experimental import pallas as pl
from jax.experimental.pallas import tpu as pltpu
from jax.experimental.pallas import tpu_sc as plsc

N, D, B, P, NB, SF, CF = 10000, 128, 1024, 6, 32, 128, 128
PB = P * B
NC, NS = 2, 16
NW = NC * NS
PAIRS_W = PB // NW
GROUP = 4
GROUPS_W = PAIRS_W // GROUP
ROWS_G = GROUP * NB


def _xw_body(x_ref, w_ref, b_ref, o_ref):
    o_ref[...] = jnp.maximum(
        jnp.dot(x_ref[...], w_ref[...], preferred_element_type=jnp.float32)
        + b_ref[...], 0.0)


def _sc_body(xw_hbm, x_hbm, nidx_hbm, pidx_hbm, hmax_out, self_out,
             nidx_v, pidx_v, nbuf0, nbuf1, selfbuf, hbuf, sem0, sem1, sem2):
    wid = lax.axis_index("s") * NC + lax.axis_index("c")
    gbase = wid * GROUPS_W
    rbase = wid * PAIRS_W

    pltpu.sync_copy(nidx_hbm.at[pl.ds(gbase, GROUPS_W)], nidx_v)
    pltpu.sync_copy(pidx_hbm.at[pl.ds(wid * 2, 2)], pidx_v)

    pltpu.async_copy(x_hbm.at[pidx_v.at[0]], selfbuf.at[pl.ds(0, 96)], sem2)
    pltpu.async_copy(x_hbm.at[pidx_v.at[1]], selfbuf.at[pl.ds(96, 96)], sem2)

    def start(g, buf, sem):
        pltpu.async_copy(xw_hbm.at[nidx_v.at[g]], buf, sem)

    def wait(buf, sem):
        pltpu.make_async_copy(xw_hbm.at[nidx_v.at[0]], buf, sem).wait()

    def reduce_group(buf, g):
        def qbody(q, carry):
            r0 = q * NB
            for c in range(D // 16):
                acc = buf[r0, pl.ds(c * 16, 16)]
                for r in range(1, NB):
                    acc = jnp.maximum(acc, buf[r0 + r, pl.ds(c * 16, 16)])
                hbuf[g * GROUP + q, pl.ds(c * 16, 16)] = acc
            return carry
        lax.fori_loop(0, GROUP, qbody, 0, unroll=False)

    start(0, nbuf0, sem0)

    def mbody(m, carry):
        g0 = 2 * m
        start(g0 + 1, nbuf1, sem1)
        wait(nbuf0, sem0)
        reduce_group(nbuf0, g0)

        @pl.when(m < GROUPS_W // 2 - 1)
        def _():
            start(g0 + 2, nbuf0, sem0)

        wait(nbuf1, sem1)
        reduce_group(nbuf1, g0 + 1)
        return carry

    lax.fori_loop(0, GROUPS_W // 2, mbody, 0, unroll=False)

    pltpu.make_async_copy(x_hbm.at[pidx_v.at[0]],
                          selfbuf.at[pl.ds(0, 96)], sem2).wait()
    pltpu.make_async_copy(x_hbm.at[pidx_v.at[1]],
                          selfbuf.at[pl.ds(96, 96)], sem2).wait()

    pltpu.sync_copy(hbuf, hmax_out.at[pl.ds(rbase, PAIRS_W)])
    pltpu.sync_copy(selfbuf, self_out.at[pl.ds(rbase, PAIRS_W)])


_sc_gather_max = functools.partial(
    pl.kernel,
    out_type=(jax.ShapeDtypeStruct((PB, D), jnp.float32),
              jax.ShapeDtypeStruct((PB, D), jnp.float32)),
    mesh=plsc.VectorSubcoreMesh(core_axis_name="c", subcore_axis_name="s"),
    scratch_types=[
        pltpu.VMEM((GROUPS_W, ROWS_G), jnp.int32),
        pltpu.VMEM((2, 96), jnp.int32),
        pltpu.VMEM((ROWS_G, D), jnp.float32),
        pltpu.VMEM((ROWS_G, D), jnp.float32),
        pltpu.VMEM((PAIRS_W, D), jnp.float32),
        pltpu.VMEM((PAIRS_W, D), jnp.float32),
        pltpu.SemaphoreType.DMA,
        pltpu.SemaphoreType.DMA,
        pltpu.SemaphoreType.DMA,
    ])(_sc_body)


def _head_body(self_ref, hmax_ref, was_ref, wah_ref, bagg_ref,
               k20_ref, k21_ref, bk2_ref, k30_ref, k31_ref, k32_ref, bk3_ref,
               w1a_ref, w1b_ref, b1_ref, w2_ref, b2_ref, w3_ref, b3_ref,
               o_ref):
    def dot(a, b):
        return jnp.dot(a, b, preferred_element_type=jnp.float32)

    steps = []
    for t in range(P):
        s = (dot(self_ref[t], was_ref[...]) + dot(hmax_ref[t], wah_ref[...])
             + bagg_ref[...])
        steps.append(jnp.maximum(s, 0.0))

    p2 = None
    for t in range(P - 1):
        c = jnp.maximum(dot(steps[t], k20_ref[...])
                        + dot(steps[t + 1], k21_ref[...]) + bk2_ref[...], 0.0)
        p2 = c if p2 is None else jnp.maximum(p2, c)

    p3 = None
    for t in range(P - 2):
        c = jnp.maximum(dot(steps[t], k30_ref[...])
                        + dot(steps[t + 1], k31_ref[...])
                        + dot(steps[t + 2], k32_ref[...]) + bk3_ref[...], 0.0)
        p3 = c if p3 is None else jnp.maximum(p3, c)

    h1 = jnp.maximum(dot(p2, w1a_ref[...]) + dot(p3, w1b_ref[...])
                     + b1_ref[...], 0.0)
    h2 = jnp.maximum(dot(h1, w2_ref[...]) + b2_ref[...], 0.0)
    logits = dot(h2, w3_ref[...]) + b3_ref[...]
    mx = jnp.max(logits, axis=-1, keepdims=True)
    e = jnp.exp(logits - mx)
    o_ref[...] = e / jnp.sum(e, axis=-1, keepdims=True)


def kernel(x, path_nodes, neighbor_idx, W_pool, b_pool, W_agg, b_agg,
           K2, bK2, K3, bK3, W1, b1, W2, b2, W3, b3):
    xw = pl.pallas_call(
        _xw_body,
        grid=(5,),
        in_specs=[pl.BlockSpec((N // 5, D), lambda i: (i, 0)),
                  pl.BlockSpec((D, D), lambda i: (0, 0)),
                  pl.BlockSpec((1, D), lambda i: (0, 0))],
        out_specs=pl.BlockSpec((N // 5, D), lambda i: (i, 0)),
        out_shape=jax.ShapeDtypeStruct((N, D), jnp.float32),
    )(x, W_pool, b_pool.reshape(1, D))

    nidx_t = jnp.transpose(neighbor_idx, (1, 0, 2)).reshape(PB * NB // ROWS_G,
                                                            ROWS_G)
    pidx_t = jnp.transpose(path_nodes, (1, 0)).reshape(NW * 2, 96)

    hmax_flat, self_flat = _sc_gather_max(xw, x, nidx_t, pidx_t)

    BBLK = 512
    full = lambda shape: pl.BlockSpec(shape, lambda i: tuple(0 for _ in shape))
    out = pl.pallas_call(
        _head_body,
        grid=(B // BBLK,),
        in_specs=[
            pl.BlockSpec((P, BBLK, D), lambda i: (0, i, 0)),
            pl.BlockSpec((P, BBLK, D), lambda i: (0, i, 0)),
            full((SF, SF)), full((SF, SF)), full((1, SF)),
            full((SF, CF)), full((SF, CF)), full((1, CF)),
            full((SF, CF)), full((SF, CF)), full((SF, CF)), full((1, CF)),
            full((CF, 400)), full((CF, 400)), full((1, 400)),
            full((400, 400)), full((1, 400)),
            full((400, 2)), full((1, 2)),
        ],
        out_specs=pl.BlockSpec((BBLK, 2), lambda i: (i, 0)),
        out_shape=jax.ShapeDtypeStruct((B, 2), jnp.float32),
    )(self_flat.reshape(P, B, D), hmax_flat.reshape(P, B, D),
      W_agg[:D], W_agg[D:], b_agg.reshape(1, SF),
      K2[0], K2[1], bK2.reshape(1, CF),
      K3[0], K3[1], K3[2], bK3.reshape(1, CF),
      W1[:CF], W1[CF:], b1.reshape(1, 400),
      W2, b2.reshape(1, 400),
      W3, b3.reshape(1, 2))
    return out


# speedup vs baseline: 3.3553x; 1.0008x over previous
import functools

import jax
import jax.numpy as jnp
from jax import lax
from jax.experimental import pallas as pl
from jax.experimental.pallas import tpu as pltpu
from jax.experimental.pallas import tpu_sc as plsc

N, D, B, P, NB, SF, CF = 10000, 128, 1024, 6, 32, 128, 128
PB = P * B
NC, NS = 2, 16
NW = NC * NS
PAIRS_W = PB // NW
GROUP = 4
GROUPS_W = PAIRS_W // GROUP
ROWS_G = GROUP * NB


def _xw_body(x_ref, w_ref, b_ref, o_ref):
    o_ref[...] = jnp.maximum(
        jnp.dot(x_ref[...], w_ref[...], preferred_element_type=jnp.float32)
        + b_ref[...], 0.0)


def _sc_body(xw_hbm, x_hbm, nidx_hbm, pidx_hbm, hmax_out, self_out,
             nidx_v, pidx_v, nbuf0, nbuf1, selfbuf, hbuf, sem0, sem1, sem2):
    wid = lax.axis_index("s") * NC + lax.axis_index("c")
    gbase = wid * GROUPS_W
    rbase = wid * PAIRS_W

    pltpu.sync_copy(nidx_hbm.at[pl.ds(gbase, GROUPS_W)], nidx_v)
    pltpu.sync_copy(pidx_hbm.at[pl.ds(wid * 2, 2)], pidx_v)

    pltpu.async_copy(x_hbm.at[pidx_v.at[0]], selfbuf.at[pl.ds(0, 96)], sem2)
    pltpu.async_copy(x_hbm.at[pidx_v.at[1]], selfbuf.at[pl.ds(96, 96)], sem2)

    def start(g, buf, sem):
        pltpu.async_copy(xw_hbm.at[nidx_v.at[g]], buf, sem)

    def wait(buf, sem):
        pltpu.make_async_copy(xw_hbm.at[nidx_v.at[0]], buf, sem).wait()

    def reduce_group(buf, g):
        def qbody(q, carry):
            r0 = q * NB
            for c in range(D // 16):
                acc = buf[r0, pl.ds(c * 16, 16)]
                for r in range(1, NB):
                    acc = jnp.maximum(acc, buf[r0 + r, pl.ds(c * 16, 16)])
                hbuf[g * GROUP + q, pl.ds(c * 16, 16)] = acc
            return carry
        lax.fori_loop(0, GROUP, qbody, 0, unroll=False)

    start(0, nbuf0, sem0)

    def mbody(m, carry):
        g0 = 2 * m
        start(g0 + 1, nbuf1, sem1)
        wait(nbuf0, sem0)
        reduce_group(nbuf0, g0)

        @pl.when(m < GROUPS_W // 2 - 1)
        def _():
            start(g0 + 2, nbuf0, sem0)

        wait(nbuf1, sem1)
        reduce_group(nbuf1, g0 + 1)
        return carry

    lax.fori_loop(0, GROUPS_W // 2, mbody, 0, unroll=False)

    pltpu.make_async_copy(x_hbm.at[pidx_v.at[0]],
                          selfbuf.at[pl.ds(0, 96)], sem2).wait()
    pltpu.make_async_copy(x_hbm.at[pidx_v.at[1]],
                          selfbuf.at[pl.ds(96, 96)], sem2).wait()

    pltpu.sync_copy(hbuf, hmax_out.at[pl.ds(rbase, PAIRS_W)])
    pltpu.sync_copy(selfbuf, self_out.at[pl.ds(rbase, PAIRS_W)])


_sc_gather_max = functools.partial(
    pl.kernel,
    out_type=(jax.ShapeDtypeStruct((PB, D), jnp.float32),
              jax.ShapeDtypeStruct((PB, D), jnp.float32)),
    mesh=plsc.VectorSubcoreMesh(core_axis_name="c", subcore_axis_name="s"),
    scratch_types=[
        pltpu.VMEM((GROUPS_W, ROWS_G), jnp.int32),
        pltpu.VMEM((2, 96), jnp.int32),
        pltpu.VMEM((ROWS_G, D), jnp.float32),
        pltpu.VMEM((ROWS_G, D), jnp.float32),
        pltpu.VMEM((PAIRS_W, D), jnp.float32),
        pltpu.VMEM((PAIRS_W, D), jnp.float32),
        pltpu.SemaphoreType.DMA,
        pltpu.SemaphoreType.DMA,
        pltpu.SemaphoreType.DMA,
    ])(_sc_body)


def _head_body(self_ref, hmax_ref, wagg_ref, bagg_ref,
               k2_ref, bk2_ref, k3_ref, bk3_ref,
               w1_ref, b1_ref, w2_ref, b2_ref, w3_ref, b3_ref,
               o_ref):
    def dot(a, b):
        return jnp.dot(a, b, preferred_element_type=jnp.float32)

    was, wah = wagg_ref[0:D], wagg_ref[D:2 * D]
    k20, k21 = k2_ref[0], k2_ref[1]
    k30, k31, k32 = k3_ref[0], k3_ref[1], k3_ref[2]
    w1a, w1b = w1_ref[0:CF], w1_ref[CF:2 * CF]

    steps = []
    for t in range(P):
        s = (dot(self_ref[t], was) + dot(hmax_ref[t], wah)
             + bagg_ref[...])
        steps.append(jnp.maximum(s, 0.0))

    p2 = None
    for t in range(P - 1):
        c = jnp.maximum(dot(steps[t], k20)
                        + dot(steps[t + 1], k21) + bk2_ref[...], 0.0)
        p2 = c if p2 is None else jnp.maximum(p2, c)

    p3 = None
    for t in range(P - 2):
        c = jnp.maximum(dot(steps[t], k30)
                        + dot(steps[t + 1], k31)
                        + dot(steps[t + 2], k32) + bk3_ref[...], 0.0)
        p3 = c if p3 is None else jnp.maximum(p3, c)

    h1 = jnp.maximum(dot(p2, w1a) + dot(p3, w1b)
                     + b1_ref[...], 0.0)
    h2 = jnp.maximum(dot(h1, w2_ref[...]) + b2_ref[...], 0.0)
    logits = dot(h2, w3_ref[...]) + b3_ref[...]
    mx = jnp.max(logits, axis=-1, keepdims=True)
    e = jnp.exp(logits - mx)
    o_ref[...] = e / jnp.sum(e, axis=-1, keepdims=True)


def kernel(x, path_nodes, neighbor_idx, W_pool, b_pool, W_agg, b_agg,
           K2, bK2, K3, bK3, W1, b1, W2, b2, W3, b3):
    xw = pl.pallas_call(
        _xw_body,
        grid=(5,),
        in_specs=[pl.BlockSpec((N // 5, D), lambda i: (i, 0)),
                  pl.BlockSpec((D, D), lambda i: (0, 0)),
                  pl.BlockSpec((1, D), lambda i: (0, 0))],
        out_specs=pl.BlockSpec((N // 5, D), lambda i: (i, 0)),
        out_shape=jax.ShapeDtypeStruct((N, D), jnp.float32),
    )(x, W_pool, b_pool.reshape(1, D))

    nidx_t = jnp.transpose(neighbor_idx, (1, 0, 2)).reshape(PB * NB // ROWS_G,
                                                            ROWS_G)
    pidx_t = jnp.transpose(path_nodes, (1, 0)).reshape(NW * 2, 96)

    hmax_flat, self_flat = _sc_gather_max(xw, x, nidx_t, pidx_t)

    BBLK = 512
    full = lambda shape: pl.BlockSpec(shape, lambda i: tuple(0 for _ in shape))
    out = pl.pallas_call(
        _head_body,
        grid=(B // BBLK,),
        in_specs=[
            pl.BlockSpec((P, BBLK, D), lambda i: (0, i, 0)),
            pl.BlockSpec((P, BBLK, D), lambda i: (0, i, 0)),
            full((2 * D, SF)), full((1, SF)),
            full((2, SF, CF)), full((1, CF)),
            full((3, SF, CF)), full((1, CF)),
            full((2 * CF, 400)), full((1, 400)),
            full((400, 400)), full((1, 400)),
            full((400, 2)), full((1, 2)),
        ],
        out_specs=pl.BlockSpec((BBLK, 2), lambda i: (i, 0)),
        out_shape=jax.ShapeDtypeStruct((B, 2), jnp.float32),
    )(self_flat.reshape(P, B, D), hmax_flat.reshape(P, B, D),
      W_agg, b_agg.reshape(1, SF),
      K2, bK2.reshape(1, CF),
      K3, bK3.reshape(1, CF),
      W1, b1.reshape(1, 400),
      W2, b2.reshape(1, 400),
      W3, b3.reshape(1, 2))
    return out
